# baseline (speedup 1.0000x reference)
import functools

import jax
import jax.numpy as jnp
from jax import lax
from jax.experimental import pallas as pl
from jax.experimental.pallas import tpu as pltpu
from jax.experimental.pallas import tpu_sc as plsc

_VOCAB = 32000
_D = 768
_MAXP = 64
_SCALE = float(_D) ** 0.5

_NC = 2
_NS = 16
_NW = _NC * _NS

_CHUNK = 16
_NBUF = 4
_LANES = 16
_VPR = _D // _LANES


def _fma_chunk(g, pos_v, b):
    poff = b * _CHUNK

    def row_body(r, carry):
        for u in range(_VPR):
            sl = pl.ds(u * _LANES, _LANES)
            g[b, r, sl] = g[b, r, sl] * _SCALE + pos_v[poff + r, sl]
        return carry

    lax.fori_loop(0, _CHUNK, row_body, 0, unroll=False)


def _body(ids_hbm, table_hbm, pos_hbm, out_hbm, idx_v, pos_v, g,
          sem_g, sem_s):
    wid = lax.axis_index("s") * _NC + lax.axis_index("c")
    rows_per_w = ids_hbm.shape[1] * ids_hbm.shape[2]
    n_chunks = ids_hbm.shape[1]
    wbase = wid * rows_per_w

    pltpu.sync_copy(ids_hbm.at[wid], idx_v)
    pltpu.sync_copy(pos_hbm, pos_v)

    def start_gather(c, b):
        pltpu.async_copy(table_hbm.at[idx_v.at[c]], g.at[b], sem_g[b])

    def start_scatter(c, b):
        pltpu.async_copy(g.at[b], out_hbm.at[pl.ds(wbase + c * _CHUNK, _CHUNK)],
                         sem_s[b])

    def wait_gather(b):
        pltpu.make_async_copy(table_hbm.at[idx_v.at[0]], g.at[b],
                              sem_g[b]).wait()

    def wait_scatter(b):
        pltpu.make_async_copy(g.at[b], out_hbm.at[pl.ds(0, _CHUNK)],
                              sem_s[b]).wait()

    for b in range(_NBUF - 1):
        start_gather(b, b)

    def process(c, b, first, last):
        wait_gather(b)
        _fma_chunk(g, pos_v, b)
        start_scatter(c, b)
        if not last:
            bn = (b + _NBUF - 1) % _NBUF
            if not first:
                wait_scatter(bn)
            start_gather(c + (_NBUF - 1), bn)

    for b in range(_NBUF):
        process(b, b, first=(b == 0), last=False)

    n_groups = n_chunks // _NBUF

    def group_body(gidx, carry):
        c0 = gidx * _NBUF
        for b in range(_NBUF):
            process(c0 + b, b, first=False, last=False)
        return carry

    lax.fori_loop(1, n_groups - 1, group_body, 0, unroll=False)

    c0 = (n_groups - 1) * _NBUF
    for b in range(_NBUF):
        process(c0 + b, b, first=False, last=(b > 0))

    for b in range(_NBUF):
        wait_scatter(b)


@jax.jit
def kernel(input_ids, token_embedding, position_embedding):
    batch, seq = input_ids.shape
    total = batch * seq
    rows_per_w = total // _NW
    n_chunks = rows_per_w // _CHUNK

    ids3 = input_ids.reshape(_NW, n_chunks, _CHUNK).astype(jnp.int32)

    mesh = plsc.VectorSubcoreMesh(core_axis_name="c", subcore_axis_name="s")
    run = pl.kernel(
        _body,
        out_type=jax.ShapeDtypeStruct((total, _D), jnp.float32),
        mesh=mesh,
        compiler_params=pltpu.CompilerParams(use_tc_tiling_on_sc=False),
        scratch_types=[
            pltpu.VMEM((n_chunks, _CHUNK), jnp.int32),
            pltpu.VMEM((_MAXP, _D), jnp.float32),
            pltpu.VMEM((_NBUF, _CHUNK, _D), jnp.float32),
            [pltpu.SemaphoreType.DMA] * _NBUF,
            [pltpu.SemaphoreType.DMA] * _NBUF,
        ],
    )
    out = run(ids3, token_embedding, position_embedding)
    return out.reshape(batch, seq, _D)
